# baseline (speedup 1.0000x reference)
import functools

import jax
import jax.numpy as jnp
from jax import lax
from jax.experimental import pallas as pl
from jax.experimental.pallas import tpu as pltpu
from jax.experimental.pallas import tpu_sc as plsc

N_NODES = 10000
N_EDGES = 320000
D_IN = 128
D_HID = 256
D_OUT = 128

NC, NS = 2, 16
NW = NC * NS
CHUNK = 80
EPW = N_EDGES // NW
NCHUNK = EPW // CHUNK

ROWS_BIG = 640
ROWS_LAST = N_NODES - 15 * ROWS_BIG
WB_CHUNK = 80

_sc_mesh = plsc.VectorSubcoreMesh(
    core_axis_name="c", subcore_axis_name="s", num_cores=NC, num_subcores=NS)


@functools.partial(
    pl.kernel,
    out_type=jax.ShapeDtypeStruct((NC * N_NODES,), jnp.float32),
    mesh=_sc_mesh,
    scratch_types=[
        pltpu.VMEM((NCHUNK, CHUNK), jnp.int32),
        pltpu.VMEM((CHUNK,), jnp.float32),
        pltpu.VMEM((ROWS_BIG,), jnp.float32),
        pltpu.VMEM_SHARED((N_NODES,), jnp.float32),
    ],
)
def _deg_kernel(dst3_hbm, out_hbm, dst_v, ones_v, stage_v, deg_sh):
    c = lax.axis_index("c")
    s = lax.axis_index("s")
    wid = s * NC + c

    def fill_body(i, _):
        ones_v[pl.ds(i * 16, 16)] = jnp.ones((16,), jnp.float32)
        return 0
    lax.fori_loop(0, CHUNK // 16, fill_body, 0)

    def zero_body(i, _):
        stage_v[pl.ds(i * 16, 16)] = jnp.zeros((16,), jnp.float32)
        return 0
    lax.fori_loop(0, ROWS_BIG // 16, zero_body, 0)

    @pl.when(s < 15)
    def _():
        pltpu.sync_copy(stage_v, deg_sh.at[pl.ds(s * ROWS_BIG, ROWS_BIG)])

    @pl.when(s == 15)
    def _():
        pltpu.sync_copy(stage_v.at[pl.ds(0, ROWS_LAST)],
                        deg_sh.at[pl.ds(s * ROWS_BIG, ROWS_LAST)])

    plsc.subcore_barrier()

    pltpu.sync_copy(dst3_hbm.at[wid], dst_v)

    def chunk_body(k, _):
        pltpu.sync_copy(ones_v, deg_sh.at[dst_v.at[k]], add=True)
        return 0
    lax.fori_loop(0, NCHUNK, chunk_body, 0)

    plsc.subcore_barrier()

    row0 = s * ROWS_BIG

    @pl.when(s < 15)
    def _():
        pltpu.sync_copy(deg_sh.at[pl.ds(row0, ROWS_BIG)], stage_v)
        pltpu.sync_copy(stage_v,
                        out_hbm.at[pl.ds(c * N_NODES + row0, ROWS_BIG)])

    @pl.when(s == 15)
    def _():
        pltpu.sync_copy(deg_sh.at[pl.ds(row0, ROWS_LAST)],
                        stage_v.at[pl.ds(0, ROWS_LAST)])
        pltpu.sync_copy(stage_v.at[pl.ds(0, ROWS_LAST)],
                        out_hbm.at[pl.ds(c * N_NODES + row0, ROWS_LAST)])


@functools.partial(
    pl.kernel,
    out_type=jax.ShapeDtypeStruct((NC, N_NODES, D_OUT), jnp.float32),
    mesh=_sc_mesh,
    scratch_types=[
        pltpu.VMEM((EPW,), jnp.int32),
        pltpu.VMEM((CHUNK, D_OUT), jnp.float32),
        pltpu.VMEM((CHUNK, D_OUT), jnp.float32),
        pltpu.VMEM((CHUNK, D_OUT), jnp.float32),
        pltpu.VMEM((CHUNK,), jnp.int32),
        pltpu.VMEM((CHUNK,), jnp.int32),
        pltpu.VMEM((CHUNK,), jnp.int32),
        pltpu.VMEM_SHARED((N_NODES, D_OUT), jnp.float32),
        pltpu.SemaphoreType.DMA,
        pltpu.SemaphoreType.DMA,
        pltpu.SemaphoreType.DMA,
        pltpu.SemaphoreType.DMA,
        pltpu.SemaphoreType.DMA,
        pltpu.SemaphoreType.DMA,
        pltpu.SemaphoreType.DMA,
        pltpu.SemaphoreType.DMA,
        pltpu.SemaphoreType.DMA,
    ],
)
def _agg_kernel(y_hbm, src_hbm, dst_hbm, out_hbm,
                src_v, rows0, rows1, rows2, db0, db1, db2, acc_sh,
                sg0, sg1, sg2, ss0, ss1, ss2, sd0, sd1, sd2):
    c = lax.axis_index("c")
    s = lax.axis_index("s")
    wid = s * NC + c

    rows = [rows0, rows1, rows2]
    db = [db0, db1, db2]
    sg = [sg0, sg1, sg2]
    ss = [ss0, ss1, ss2]
    sd = [sd0, sd1, sd2]

    row0 = s * ROWS_BIG
    nwb = jnp.where(s < 15, ROWS_BIG // WB_CHUNK, ROWS_LAST // WB_CHUNK)

    def zrow(i, _):
        def zlane(j, _):
            rows0[i, pl.ds(j * 16, 16)] = jnp.zeros((16,), jnp.float32)
            return 0
        lax.fori_loop(0, D_OUT // 16, zlane, 0)
        return 0
    lax.fori_loop(0, WB_CHUNK, zrow, 0)

    def zb(k, _):
        pltpu.sync_copy(rows0,
                        acc_sh.at[pl.ds(row0 + k * WB_CHUNK, WB_CHUNK)])
        return 0
    lax.fori_loop(0, nwb, zb, 0)

    plsc.subcore_barrier()

    pltpu.sync_copy(src_hbm.at[pl.ds(wid * EPW, EPW)], src_v)
    base = wid * EPW

    def sidx(k):
        return src_v.at[pl.ds(k * CHUNK, CHUNK)]

    def didx(k):
        return dst_hbm.at[pl.ds(base + k * CHUNK, CHUNK)]

    def fire(k, j):
        pltpu.async_copy(y_hbm.at[sidx(k)], rows[j], sg[j])
        pltpu.async_copy(didx(k), db[j], sd[j])

    def consume(k, j):
        pltpu.make_async_copy(y_hbm.at[sidx(k)], rows[j], sg[j]).wait()
        pltpu.make_async_copy(didx(k), db[j], sd[j]).wait()
        pltpu.async_copy(rows[j], acc_sh.at[db[j]], ss[j], add=True)

    def drain(j):
        pltpu.make_async_copy(rows[j], acc_sh.at[db[j]], ss[j]).wait()

    fire(0, 0)
    fire(1, 1)
    fire(2, 2)

    def group(i, _):
        k0 = 3 * i
        consume(k0, 0)
        consume(k0 + 1, 1)
        consume(k0 + 2, 2)
        drain(0)
        fire(k0 + 3, 0)
        drain(1)
        fire(k0 + 4, 1)
        drain(2)

        @pl.when(k0 + 5 < NCHUNK)
        def _():
            fire(k0 + 5, 2)
        return 0
    lax.fori_loop(0, NCHUNK // 3, group, 0)

    consume(NCHUNK - 2, 0)
    consume(NCHUNK - 1, 1)
    drain(0)
    drain(1)

    plsc.subcore_barrier()

    def wb(k, _):
        r = row0 + k * WB_CHUNK
        pltpu.sync_copy(acc_sh.at[pl.ds(r, WB_CHUNK)],
                        out_hbm.at[c, pl.ds(r, WB_CHUNK)])
        return 0
    lax.fori_loop(0, nwb, wb, 0)


_R = 1000


def _dis(degT_ref):
    deg = jnp.sum(degT_ref[...], axis=1) + 1.0
    return lax.rsqrt(deg)[:, None]


def _tc_a_body(x_ref, w1_ref, degT_ref, ylo_ref, yhi_ref):
    dis = _dis(degT_ref)
    xw = jnp.dot(x_ref[...], w1_ref[...], preferred_element_type=jnp.float32)
    y = xw * dis
    ylo_ref[...] = y[:, :D_OUT]
    yhi_ref[...] = y[:, D_OUT:]


def _tc_b_body(plo_ref, phi_ref, ylo_ref, yhi_ref, degT_ref, b1_ref, a_ref,
               w2_ref, y2_ref):
    dis = _dis(degT_ref)
    hlo = dis * (plo_ref[0] + plo_ref[1] + ylo_ref[...]) + b1_ref[:, :D_OUT]
    hhi = dis * (phi_ref[0] + phi_ref[1] + yhi_ref[...]) + b1_ref[:, D_OUT:]
    h = jnp.concatenate([hlo, hhi], axis=1)
    a = a_ref[0, 0]
    h = jnp.where(h >= 0, h, a * h)
    y2 = jnp.dot(h, w2_ref[...], preferred_element_type=jnp.float32)
    y2_ref[...] = y2 * dis


def _tc_c_body(p2_ref, y2_ref, degT_ref, b2_ref, a_ref, wp1_ref, bp1_ref,
               wp2_ref, bp2_ref, out_ref):
    dis = _dis(degT_ref)
    a = a_ref[0, 0]
    h = dis * (p2_ref[0] + p2_ref[1] + y2_ref[...]) + b2_ref[...]
    h = jnp.where(h >= 0, h, a * h)
    hid = jnp.dot(h, wp1_ref[...], preferred_element_type=jnp.float32)
    hid = hid + bp1_ref[...]
    hid = jnp.where(hid > 0, hid, jnp.exp(hid) - 1.0)
    out = jnp.dot(hid, wp2_ref[...], preferred_element_type=jnp.float32)
    out_ref[...] = out + bp2_ref[...]


def _row_spec(d):
    return pl.BlockSpec((_R, d), lambda i: (i, 0))


def _full_spec(shape):
    nd = len(shape)
    return pl.BlockSpec(shape, lambda i: (0,) * nd)


def _part_spec(d):
    return pl.BlockSpec((NC, _R, d), lambda i: (0, i, 0))


def kernel(node_features, edge_index, W1, b1, W2, b2, prelu_a, Wp1, bp1,
           Wp2, bp2):
    src = edge_index[0]
    dst = edge_index[1]
    dst3 = dst.reshape(NW, NCHUNK, CHUNK)

    deg_parts = _deg_kernel(dst3).reshape(NC, N_NODES)
    degT = deg_parts.T

    b1r = b1.reshape(1, D_HID)
    b2r = b2.reshape(1, D_OUT)
    bp1r = bp1.reshape(1, D_OUT)
    bp2r = bp2.reshape(1, D_OUT)
    ar = prelu_a.reshape(1, 1)

    grid = (N_NODES // _R,)

    y1_lo, y1_hi = pl.pallas_call(
        _tc_a_body,
        grid=grid,
        in_specs=[_row_spec(D_IN), _full_spec((D_IN, D_HID)), _row_spec(NC)],
        out_specs=[_row_spec(D_OUT), _row_spec(D_OUT)],
        out_shape=[jax.ShapeDtypeStruct((N_NODES, D_OUT), jnp.float32),
                   jax.ShapeDtypeStruct((N_NODES, D_OUT), jnp.float32)],
    )(node_features, W1, degT)

    p_lo = _agg_kernel(y1_lo, src, dst)
    p_hi = _agg_kernel(y1_hi, src, dst)

    y2 = pl.pallas_call(
        _tc_b_body,
        grid=grid,
        in_specs=[_part_spec(D_OUT), _part_spec(D_OUT),
                  _row_spec(D_OUT), _row_spec(D_OUT), _row_spec(NC),
                  _full_spec((1, D_HID)), _full_spec((1, 1)),
                  _full_spec((D_HID, D_OUT))],
        out_specs=_row_spec(D_OUT),
        out_shape=jax.ShapeDtypeStruct((N_NODES, D_OUT), jnp.float32),
    )(p_lo, p_hi, y1_lo, y1_hi, degT, b1r, ar, W2)

    p2 = _agg_kernel(y2, src, dst)

    out = pl.pallas_call(
        _tc_c_body,
        grid=grid,
        in_specs=[_part_spec(D_OUT), _row_spec(D_OUT), _row_spec(NC),
                  _full_spec((1, D_OUT)), _full_spec((1, 1)),
                  _full_spec((D_OUT, D_OUT)), _full_spec((1, D_OUT)),
                  _full_spec((D_OUT, D_OUT)), _full_spec((1, D_OUT))],
        out_specs=_row_spec(D_OUT),
        out_shape=jax.ShapeDtypeStruct((N_NODES, D_OUT), jnp.float32),
    )(p2, y2, degT, b2r, ar, Wp1, bp1r, Wp2, bp2r)

    return out


# speedup vs baseline: 1.0239x; 1.0239x over previous
import functools

import jax
import jax.numpy as jnp
from jax import lax
from jax.experimental import pallas as pl
from jax.experimental.pallas import tpu as pltpu
from jax.experimental.pallas import tpu_sc as plsc

N_NODES = 10000
N_EDGES = 320000
D_IN = 128
D_HID = 256
D_OUT = 128

NC, NS = 2, 16
NW = NC * NS
CHUNK = 80
EPW = N_EDGES // NW
NCHUNK = EPW // CHUNK

ROWS_BIG = 640
ROWS_LAST = N_NODES - 15 * ROWS_BIG
WB_CHUNK = 80

_sc_mesh = plsc.VectorSubcoreMesh(
    core_axis_name="c", subcore_axis_name="s", num_cores=NC, num_subcores=NS)


@functools.partial(
    pl.kernel,
    out_type=jax.ShapeDtypeStruct((NC * N_NODES,), jnp.float32),
    mesh=_sc_mesh,
    scratch_types=[
        pltpu.VMEM((NCHUNK, CHUNK), jnp.int32),
        pltpu.VMEM((CHUNK,), jnp.float32),
        pltpu.VMEM((ROWS_BIG,), jnp.float32),
        pltpu.VMEM_SHARED((N_NODES,), jnp.float32),
    ],
)
def _deg_kernel(dst3_hbm, out_hbm, dst_v, ones_v, stage_v, deg_sh):
    c = lax.axis_index("c")
    s = lax.axis_index("s")
    wid = s * NC + c

    def fill_body(i, _):
        ones_v[pl.ds(i * 16, 16)] = jnp.ones((16,), jnp.float32)
        return 0
    lax.fori_loop(0, CHUNK // 16, fill_body, 0)

    def zero_body(i, _):
        stage_v[pl.ds(i * 16, 16)] = jnp.zeros((16,), jnp.float32)
        return 0
    lax.fori_loop(0, ROWS_BIG // 16, zero_body, 0)

    @pl.when(s < 15)
    def _():
        pltpu.sync_copy(stage_v, deg_sh.at[pl.ds(s * ROWS_BIG, ROWS_BIG)])

    @pl.when(s == 15)
    def _():
        pltpu.sync_copy(stage_v.at[pl.ds(0, ROWS_LAST)],
                        deg_sh.at[pl.ds(s * ROWS_BIG, ROWS_LAST)])

    plsc.subcore_barrier()

    pltpu.sync_copy(dst3_hbm.at[wid], dst_v)

    def chunk_body(k, _):
        pltpu.sync_copy(ones_v, deg_sh.at[dst_v.at[k]], add=True)
        return 0
    lax.fori_loop(0, NCHUNK, chunk_body, 0)

    plsc.subcore_barrier()

    row0 = s * ROWS_BIG

    @pl.when(s < 15)
    def _():
        pltpu.sync_copy(deg_sh.at[pl.ds(row0, ROWS_BIG)], stage_v)
        pltpu.sync_copy(stage_v,
                        out_hbm.at[pl.ds(c * N_NODES + row0, ROWS_BIG)])

    @pl.when(s == 15)
    def _():
        pltpu.sync_copy(deg_sh.at[pl.ds(row0, ROWS_LAST)],
                        stage_v.at[pl.ds(0, ROWS_LAST)])
        pltpu.sync_copy(stage_v.at[pl.ds(0, ROWS_LAST)],
                        out_hbm.at[pl.ds(c * N_NODES + row0, ROWS_LAST)])


_AGG_SCRATCH = [
    pltpu.VMEM((EPW,), jnp.int32),
    pltpu.VMEM((CHUNK, D_OUT), jnp.float32),
    pltpu.VMEM((CHUNK, D_OUT), jnp.float32),
    pltpu.VMEM((CHUNK, D_OUT), jnp.float32),
    pltpu.VMEM((CHUNK,), jnp.int32),
    pltpu.VMEM((CHUNK,), jnp.int32),
    pltpu.VMEM((CHUNK,), jnp.int32),
    pltpu.VMEM_SHARED((N_NODES, D_OUT), jnp.float32),
    pltpu.SemaphoreType.DMA,
    pltpu.SemaphoreType.DMA,
    pltpu.SemaphoreType.DMA,
    pltpu.SemaphoreType.DMA,
    pltpu.SemaphoreType.DMA,
    pltpu.SemaphoreType.DMA,
    pltpu.SemaphoreType.DMA,
    pltpu.SemaphoreType.DMA,
    pltpu.SemaphoreType.DMA,
]


def _agg_pass(y_hbm, dst_hbm, out_ref_fn, src_v, rows, db, acc_sh,
              sg, ss, sd, s, wid):
    rows0 = rows[0]
    row0 = s * ROWS_BIG
    nwb = jnp.where(s < 15, ROWS_BIG // WB_CHUNK, ROWS_LAST // WB_CHUNK)

    def zrow(i, _):
        def zlane(j, _):
            rows0[i, pl.ds(j * 16, 16)] = jnp.zeros((16,), jnp.float32)
            return 0
        lax.fori_loop(0, D_OUT // 16, zlane, 0)
        return 0
    lax.fori_loop(0, WB_CHUNK, zrow, 0)

    def zb(k, _):
        pltpu.sync_copy(rows0,
                        acc_sh.at[pl.ds(row0 + k * WB_CHUNK, WB_CHUNK)])
        return 0
    lax.fori_loop(0, nwb, zb, 0)

    plsc.subcore_barrier()

    base = wid * EPW

    def sidx(k):
        return src_v.at[pl.ds(k * CHUNK, CHUNK)]

    def didx(k):
        return dst_hbm.at[pl.ds(base + k * CHUNK, CHUNK)]

    def fire(k, j):
        pltpu.async_copy(y_hbm.at[sidx(k)], rows[j], sg[j])
        pltpu.async_copy(didx(k), db[j], sd[j])

    def consume(k, j):
        pltpu.make_async_copy(y_hbm.at[sidx(k)], rows[j], sg[j]).wait()
        pltpu.make_async_copy(didx(k), db[j], sd[j]).wait()
        pltpu.async_copy(rows[j], acc_sh.at[db[j]], ss[j], add=True)

    def drain(j):
        pltpu.make_async_copy(rows[j], acc_sh.at[db[j]], ss[j]).wait()

    fire(0, 0)
    fire(1, 1)
    fire(2, 2)

    def group(i, _):
        k0 = 3 * i
        consume(k0, 0)
        consume(k0 + 1, 1)
        consume(k0 + 2, 2)
        drain(0)
        fire(k0 + 3, 0)
        drain(1)
        fire(k0 + 4, 1)
        drain(2)

        @pl.when(k0 + 5 < NCHUNK)
        def _():
            fire(k0 + 5, 2)
        return 0
    lax.fori_loop(0, NCHUNK // 3, group, 0)

    consume(NCHUNK - 2, 0)
    consume(NCHUNK - 1, 1)
    drain(0)
    drain(1)

    plsc.subcore_barrier()

    def wb(k, _):
        r = row0 + k * WB_CHUNK
        pltpu.sync_copy(acc_sh.at[pl.ds(r, WB_CHUNK)], out_ref_fn(r))
        return 0
    lax.fori_loop(0, nwb, wb, 0)


@functools.partial(
    pl.kernel,
    out_type=jax.ShapeDtypeStruct((NC, N_NODES, D_OUT), jnp.float32),
    mesh=_sc_mesh,
    scratch_types=_AGG_SCRATCH,
)
def _agg_kernel(y_hbm, src_hbm, dst_hbm, out_hbm,
                src_v, rows0, rows1, rows2, db0, db1, db2, acc_sh,
                sg0, sg1, sg2, ss0, ss1, ss2, sd0, sd1, sd2):
    c = lax.axis_index("c")
    s = lax.axis_index("s")
    wid = s * NC + c
    pltpu.sync_copy(src_hbm.at[pl.ds(wid * EPW, EPW)], src_v)
    _agg_pass(y_hbm, dst_hbm,
              lambda r: out_hbm.at[c, pl.ds(r, WB_CHUNK)],
              src_v, [rows0, rows1, rows2], [db0, db1, db2], acc_sh,
              [sg0, sg1, sg2], [ss0, ss1, ss2], [sd0, sd1, sd2], s, wid)


@functools.partial(
    pl.kernel,
    out_type=jax.ShapeDtypeStruct((2, NC, N_NODES, D_OUT), jnp.float32),
    mesh=_sc_mesh,
    scratch_types=_AGG_SCRATCH,
)
def _agg2_kernel(ylo_hbm, yhi_hbm, src_hbm, dst_hbm, out_hbm,
                 src_v, rows0, rows1, rows2, db0, db1, db2, acc_sh,
                 sg0, sg1, sg2, ss0, ss1, ss2, sd0, sd1, sd2):
    c = lax.axis_index("c")
    s = lax.axis_index("s")
    wid = s * NC + c
    pltpu.sync_copy(src_hbm.at[pl.ds(wid * EPW, EPW)], src_v)
    for h, y_hbm in enumerate((ylo_hbm, yhi_hbm)):
        _agg_pass(y_hbm, dst_hbm,
                  lambda r, h=h: out_hbm.at[h, c, pl.ds(r, WB_CHUNK)],
                  src_v, [rows0, rows1, rows2], [db0, db1, db2], acc_sh,
                  [sg0, sg1, sg2], [ss0, ss1, ss2], [sd0, sd1, sd2], s, wid)


_R = 2000


def _dis(degT_ref):
    deg = jnp.sum(degT_ref[...], axis=1) + 1.0
    return lax.rsqrt(deg)[:, None]


def _tc_a_body(x_ref, w1_ref, degT_ref, ylo_ref, yhi_ref):
    dis = _dis(degT_ref)
    xw = jnp.dot(x_ref[...], w1_ref[...], preferred_element_type=jnp.float32)
    y = xw * dis
    ylo_ref[...] = y[:, :D_OUT]
    yhi_ref[...] = y[:, D_OUT:]


def _tc_b_body(p4_ref, ylo_ref, yhi_ref, degT_ref, b1_ref, a_ref,
               w2_ref, y2_ref):
    dis = _dis(degT_ref)
    hlo = dis * (p4_ref[0, 0] + p4_ref[0, 1] + ylo_ref[...]) + b1_ref[:, :D_OUT]
    hhi = dis * (p4_ref[1, 0] + p4_ref[1, 1] + yhi_ref[...]) + b1_ref[:, D_OUT:]
    h = jnp.concatenate([hlo, hhi], axis=1)
    a = a_ref[0, 0]
    h = jnp.where(h >= 0, h, a * h)
    y2 = jnp.dot(h, w2_ref[...], preferred_element_type=jnp.float32)
    y2_ref[...] = y2 * dis


def _tc_c_body(p2_ref, y2_ref, degT_ref, b2_ref, a_ref, wp1_ref, bp1_ref,
               wp2_ref, bp2_ref, out_ref):
    dis = _dis(degT_ref)
    a = a_ref[0, 0]
    h = dis * (p2_ref[0] + p2_ref[1] + y2_ref[...]) + b2_ref[...]
    h = jnp.where(h >= 0, h, a * h)
    hid = jnp.dot(h, wp1_ref[...], preferred_element_type=jnp.float32)
    hid = hid + bp1_ref[...]
    hid = jnp.where(hid > 0, hid, jnp.exp(hid) - 1.0)
    out = jnp.dot(hid, wp2_ref[...], preferred_element_type=jnp.float32)
    out_ref[...] = out + bp2_ref[...]


def _row_spec(d):
    return pl.BlockSpec((_R, d), lambda i: (i, 0))


def _full_spec(shape):
    nd = len(shape)
    return pl.BlockSpec(shape, lambda i: (0,) * nd)


def _part_spec(d):
    return pl.BlockSpec((NC, _R, d), lambda i: (0, i, 0))


def kernel(node_features, edge_index, W1, b1, W2, b2, prelu_a, Wp1, bp1,
           Wp2, bp2):
    src = edge_index[0]
    dst = edge_index[1]
    dst3 = dst.reshape(NW, NCHUNK, CHUNK)

    deg_parts = _deg_kernel(dst3).reshape(NC, N_NODES)
    degT = deg_parts.T

    b1r = b1.reshape(1, D_HID)
    b2r = b2.reshape(1, D_OUT)
    bp1r = bp1.reshape(1, D_OUT)
    bp2r = bp2.reshape(1, D_OUT)
    ar = prelu_a.reshape(1, 1)

    grid = (N_NODES // _R,)

    y1_lo, y1_hi = pl.pallas_call(
        _tc_a_body,
        grid=grid,
        in_specs=[_row_spec(D_IN), _full_spec((D_IN, D_HID)), _row_spec(NC)],
        out_specs=[_row_spec(D_OUT), _row_spec(D_OUT)],
        out_shape=[jax.ShapeDtypeStruct((N_NODES, D_OUT), jnp.float32),
                   jax.ShapeDtypeStruct((N_NODES, D_OUT), jnp.float32)],
    )(node_features, W1, degT)

    p4 = _agg2_kernel(y1_lo, y1_hi, src, dst)

    y2 = pl.pallas_call(
        _tc_b_body,
        grid=grid,
        in_specs=[pl.BlockSpec((2, NC, _R, D_OUT), lambda i: (0, 0, i, 0)),
                  _row_spec(D_OUT), _row_spec(D_OUT), _row_spec(NC),
                  _full_spec((1, D_HID)), _full_spec((1, 1)),
                  _full_spec((D_HID, D_OUT))],
        out_specs=_row_spec(D_OUT),
        out_shape=jax.ShapeDtypeStruct((N_NODES, D_OUT), jnp.float32),
    )(p4, y1_lo, y1_hi, degT, b1r, ar, W2)

    p2 = _agg_kernel(y2, src, dst)

    out = pl.pallas_call(
        _tc_c_body,
        grid=grid,
        in_specs=[_part_spec(D_OUT), _row_spec(D_OUT), _row_spec(NC),
                  _full_spec((1, D_OUT)), _full_spec((1, 1)),
                  _full_spec((D_OUT, D_OUT)), _full_spec((1, D_OUT)),
                  _full_spec((D_OUT, D_OUT)), _full_spec((1, D_OUT))],
        out_specs=_row_spec(D_OUT),
        out_shape=jax.ShapeDtypeStruct((N_NODES, D_OUT), jnp.float32),
    )(p2, y2, degT, b2r, ar, Wp1, bp1r, Wp2, bp2r)

    return out
